# baseline (device time: 28186 ns/iter reference)
import jax
import jax.numpy as jnp
from jax import lax
from jax.experimental import pallas as pl
from jax.experimental.pallas import tpu as pltpu

C = 16
H = C // 2


def kernel(x):
    m, n = x.shape
    half = n // 2
    rows_half = m // 2
    ch = rows_half // C

    def body(x_ref, out_ref, send_buf, x_send_sems, x_recv_sems,
             d_send_sems, d_recv_sems, y_send_sems, y_recv_sems):
        my_x = lax.axis_index("x")
        my_y = lax.axis_index("y")
        my_z = lax.axis_index("z")
        ox = 1 - my_x
        oy = 1 - my_y
        xp = (ox, my_y, my_z)
        yp = (my_x, oy, my_z)
        dp = (ox, oy, my_z)

        barrier = pltpu.get_barrier_semaphore()
        for nbr in (xp, yp, dp):
            pl.semaphore_signal(
                barrier, inc=1,
                device_id=nbr, device_id_type=pl.DeviceIdType.MESH,
            )
        pl.semaphore_wait(barrier, 3)

        send_base = my_y * rows_half
        x_dst_base = my_x * m + my_y * rows_half
        fwd_base = ox * m + my_y * rows_half

        x_rdmas = []
        d_rdmas = []
        y_rdmas = []
        for c in range(C):
            sl = slice(c * ch, (c + 1) * ch)
            x_rdmas.append(pltpu.make_async_remote_copy(
                src_ref=send_buf.at[sl, :],
                dst_ref=out_ref.at[pl.ds(x_dst_base + c * ch, ch), :],
                send_sem=x_send_sems.at[c],
                recv_sem=x_recv_sems.at[c],
                device_id=xp,
                device_id_type=pl.DeviceIdType.MESH,
            ))
        for c in range(H):
            sl = slice(c * ch, (c + 1) * ch)
            d_rdmas.append(pltpu.make_async_remote_copy(
                src_ref=send_buf.at[sl, :],
                dst_ref=out_ref.at[pl.ds(x_dst_base + c * ch, ch), :],
                send_sem=d_send_sems.at[c],
                recv_sem=d_recv_sems.at[c],
                device_id=dp,
                device_id_type=pl.DeviceIdType.MESH,
            ))
            fwd = out_ref.at[pl.ds(fwd_base + (H + c) * ch, ch), :]
            y_rdmas.append(pltpu.make_async_remote_copy(
                src_ref=fwd,
                dst_ref=fwd,
                send_sem=y_send_sems.at[c],
                recv_sem=y_recv_sems.at[c],
                device_id=yp,
                device_id_type=pl.DeviceIdType.MESH,
            ))

        for c in range(C):
            @pl.when(my_x == 0)
            def _():
                send_buf[c * ch:(c + 1) * ch, :] = x_ref[
                    pl.ds(send_base + c * ch, ch), half:]

            @pl.when(my_x == 1)
            def _():
                send_buf[c * ch:(c + 1) * ch, :] = x_ref[
                    pl.ds(send_base + c * ch, ch), :half]

            x_rdmas[c].start()
            if c < H:
                d_rdmas[c].start()

        @pl.when(my_x == 0)
        def _():
            out_ref[:m, :] = x_ref[:, :half]

        @pl.when(my_x == 1)
        def _():
            out_ref[m:, :] = x_ref[:, half:]

        for c in range(C):
            x_rdmas[c].wait_recv()
            if c >= H:
                y_rdmas[c - H].start()

        for c in range(C):
            x_rdmas[c].wait_send()
        for c in range(H):
            d_rdmas[c].wait()
            y_rdmas[c].wait()

    return pl.pallas_call(
        body,
        out_shape=jax.ShapeDtypeStruct((2 * m, half), x.dtype),
        in_specs=[pl.BlockSpec(memory_space=pltpu.VMEM)],
        out_specs=pl.BlockSpec(memory_space=pltpu.VMEM),
        scratch_shapes=[
            pltpu.VMEM((rows_half, half), x.dtype),
            pltpu.SemaphoreType.DMA((C,)),
            pltpu.SemaphoreType.DMA((C,)),
            pltpu.SemaphoreType.DMA((H,)),
            pltpu.SemaphoreType.DMA((H,)),
            pltpu.SemaphoreType.DMA((H,)),
            pltpu.SemaphoreType.DMA((H,)),
        ],
        compiler_params=pltpu.CompilerParams(collective_id=0),
    )(x)


# device time: 22737 ns/iter; 1.2397x vs baseline; 1.2397x over previous
import jax
import jax.numpy as jnp
from jax import lax
from jax.experimental import pallas as pl
from jax.experimental.pallas import tpu as pltpu

C = 32


def kernel(x):
    m, n = x.shape
    half = n // 2
    rows_half = m // 2
    ch = rows_half // C

    def body(x_ref, out_ref, send_buf, x_send_sems, x_recv_sems, y_send_sems,
             y_recv_sems):
        my_x = lax.axis_index("x")
        my_y = lax.axis_index("y")
        my_z = lax.axis_index("z")
        ox = 1 - my_x
        oy = 1 - my_y
        xp = (ox, my_y, my_z)
        yp = (my_x, oy, my_z)

        barrier = pltpu.get_barrier_semaphore()
        for nbr in (xp, yp):
            pl.semaphore_signal(
                barrier, inc=1,
                device_id=nbr, device_id_type=pl.DeviceIdType.MESH,
            )
        pl.semaphore_wait(barrier, 2)

        send_base = my_y * rows_half
        x_dst_base = my_x * m + my_y * rows_half
        fwd_base = ox * m + my_y * rows_half

        x_rdmas = []
        y_rdmas = []
        for c in range(C):
            x_rdmas.append(pltpu.make_async_remote_copy(
                src_ref=send_buf.at[c * ch:(c + 1) * ch, :],
                dst_ref=out_ref.at[pl.ds(x_dst_base + c * ch, ch), :],
                send_sem=x_send_sems.at[c],
                recv_sem=x_recv_sems.at[c],
                device_id=xp,
                device_id_type=pl.DeviceIdType.MESH,
            ))
            fwd = out_ref.at[pl.ds(fwd_base + c * ch, ch), :]
            y_rdmas.append(pltpu.make_async_remote_copy(
                src_ref=fwd,
                dst_ref=fwd,
                send_sem=y_send_sems.at[c],
                recv_sem=y_recv_sems.at[c],
                device_id=yp,
                device_id_type=pl.DeviceIdType.MESH,
            ))

        for c in range(C):
            @pl.when(my_x == 0)
            def _():
                send_buf[c * ch:(c + 1) * ch, :] = x_ref[
                    pl.ds(send_base + c * ch, ch), half:]

            @pl.when(my_x == 1)
            def _():
                send_buf[c * ch:(c + 1) * ch, :] = x_ref[
                    pl.ds(send_base + c * ch, ch), :half]

            x_rdmas[c].start()

        @pl.when(my_x == 0)
        def _():
            out_ref[:m, :] = x_ref[:, :half]

        @pl.when(my_x == 1)
        def _():
            out_ref[m:, :] = x_ref[:, half:]

        for c in range(C):
            x_rdmas[c].wait_recv()
            y_rdmas[c].start()

        for c in range(C):
            x_rdmas[c].wait_send()
            y_rdmas[c].wait()

    return pl.pallas_call(
        body,
        out_shape=jax.ShapeDtypeStruct((2 * m, half), x.dtype),
        in_specs=[pl.BlockSpec(memory_space=pltpu.VMEM)],
        out_specs=pl.BlockSpec(memory_space=pltpu.VMEM),
        scratch_shapes=[
            pltpu.VMEM((rows_half, half), x.dtype),
            pltpu.SemaphoreType.DMA((C,)),
            pltpu.SemaphoreType.DMA((C,)),
            pltpu.SemaphoreType.DMA((C,)),
            pltpu.SemaphoreType.DMA((C,)),
        ],
        compiler_params=pltpu.CompilerParams(collective_id=0),
    )(x)


# device time: 22522 ns/iter; 1.2515x vs baseline; 1.0095x over previous
import jax
import jax.numpy as jnp
from jax import lax
from jax.experimental import pallas as pl
from jax.experimental.pallas import tpu as pltpu

C = 16


def kernel(x):
    m, n = x.shape
    half = n // 2
    rows_half = m // 2
    ch = rows_half // C

    def body(x_ref, out_ref, send_buf, x_send_sems, x_recv_sems, y_send_sems,
             y_recv_sems):
        my_x = lax.axis_index("x")
        my_y = lax.axis_index("y")
        my_z = lax.axis_index("z")
        ox = 1 - my_x
        oy = 1 - my_y
        xp = (ox, my_y, my_z)
        yp = (my_x, oy, my_z)

        barrier = pltpu.get_barrier_semaphore()
        for nbr in (xp, yp):
            pl.semaphore_signal(
                barrier, inc=1,
                device_id=nbr, device_id_type=pl.DeviceIdType.MESH,
            )
        pl.semaphore_wait(barrier, 2)

        send_base = my_y * rows_half
        x_dst_base = my_x * m + my_y * rows_half
        fwd_base = ox * m + my_y * rows_half

        x_rdmas = []
        y_rdmas = []
        for c in range(C):
            x_rdmas.append(pltpu.make_async_remote_copy(
                src_ref=send_buf.at[c * ch:(c + 1) * ch, :],
                dst_ref=out_ref.at[pl.ds(x_dst_base + c * ch, ch), :],
                send_sem=x_send_sems.at[c],
                recv_sem=x_recv_sems.at[c],
                device_id=xp,
                device_id_type=pl.DeviceIdType.MESH,
            ))
            fwd = out_ref.at[pl.ds(fwd_base + c * ch, ch), :]
            y_rdmas.append(pltpu.make_async_remote_copy(
                src_ref=fwd,
                dst_ref=fwd,
                send_sem=y_send_sems.at[c],
                recv_sem=y_recv_sems.at[c],
                device_id=yp,
                device_id_type=pl.DeviceIdType.MESH,
            ))

        for c in range(C):
            @pl.when(my_x == 0)
            def _():
                send_buf[c * ch:(c + 1) * ch, :] = x_ref[
                    pl.ds(send_base + c * ch, ch), half:]

            @pl.when(my_x == 1)
            def _():
                send_buf[c * ch:(c + 1) * ch, :] = x_ref[
                    pl.ds(send_base + c * ch, ch), :half]

            x_rdmas[c].start()

        @pl.when(my_x == 0)
        def _():
            out_ref[:m, :] = x_ref[:, :half]

        @pl.when(my_x == 1)
        def _():
            out_ref[m:, :] = x_ref[:, half:]

        for c in range(C):
            x_rdmas[c].wait_recv()
            y_rdmas[c].start()

        for c in range(C):
            x_rdmas[c].wait_send()
            y_rdmas[c].wait()

    return pl.pallas_call(
        body,
        out_shape=jax.ShapeDtypeStruct((2 * m, half), x.dtype),
        in_specs=[pl.BlockSpec(memory_space=pltpu.VMEM)],
        out_specs=pl.BlockSpec(memory_space=pltpu.VMEM),
        scratch_shapes=[
            pltpu.VMEM((rows_half, half), x.dtype),
            pltpu.SemaphoreType.DMA((C,)),
            pltpu.SemaphoreType.DMA((C,)),
            pltpu.SemaphoreType.DMA((C,)),
            pltpu.SemaphoreType.DMA((C,)),
        ],
        compiler_params=pltpu.CompilerParams(collective_id=0),
    )(x)
